# baseline (speedup 1.0000x reference)
import functools

import jax
import jax.numpy as jnp
from jax import lax
from jax.experimental import pallas as pl
from jax.experimental.pallas import tpu as pltpu
from jax.experimental.pallas import tpu_sc as plsc

E = 320000
D = 128
R = 512

B = 5000

NW = 32
BPW = E // NW
C = 40
NPAIR = BPW // (2 * C)


def _sc_gather_body(table_hbm, idx_hbm, out_hbm, idx_v, rows0, rows1,
                    sem0, sem1):
    wid = lax.axis_index("s") * 2 + lax.axis_index("c")
    base = wid * BPW
    pltpu.sync_copy(idx_hbm.at[pl.ds(base, BPW)], idx_v)

    pltpu.async_copy(table_hbm.at[idx_v.at[pl.ds(0, C)]], rows0, sem0)

    @pl.loop(0, NPAIR)
    def _pair(p):
        c0 = 2 * p
        c1 = c0 + 1
        pltpu.make_async_copy(
            table_hbm.at[idx_v.at[pl.ds(c0 * C, C)]], rows0, sem0).wait()
        pltpu.async_copy(table_hbm.at[idx_v.at[pl.ds(c1 * C, C)]], rows1, sem1)
        pltpu.sync_copy(rows0, out_hbm.at[pl.ds(base + c0 * C, C)])
        pltpu.make_async_copy(
            table_hbm.at[idx_v.at[pl.ds(c1 * C, C)]], rows1, sem1).wait()
        nxt = jnp.minimum(c0 + 2, 2 * NPAIR - 2)
        pltpu.async_copy(table_hbm.at[idx_v.at[pl.ds(nxt * C, C)]], rows0, sem0)
        pltpu.sync_copy(rows1, out_hbm.at[pl.ds(base + c1 * C, C)])

    pltpu.make_async_copy(
        table_hbm.at[idx_v.at[pl.ds((2 * NPAIR - 2) * C, C)]], rows0,
        sem0).wait()


def _sc_gather(table, idx):
    mesh = plsc.VectorSubcoreMesh(core_axis_name="c", subcore_axis_name="s")
    return pl.kernel(
        _sc_gather_body,
        out_type=jax.ShapeDtypeStruct((E, D), jnp.float32),
        mesh=mesh,
        scratch_types=[
            pltpu.VMEM((BPW,), jnp.int32),
            pltpu.VMEM((C, D), jnp.float32),
            pltpu.VMEM((C, D), jnp.float32),
            pltpu.SemaphoreType.DMA,
            pltpu.SemaphoreType.DMA,
        ],
    )(table, idx)


def _tc_score_kernel(idx_ref, table_ref, n1_ref, n2_ref, hn_ref, tn_ref,
                     pos_ref, nh_ref, nt_ref):
    idx = idx_ref[:, 0]
    iota = lax.broadcasted_iota(jnp.int32, (B, R), 1)
    onehot = (iota == idx[:, None]).astype(jnp.float32)
    r = jnp.dot(onehot, table_ref[...], preferred_element_type=jnp.float32)

    n1 = n1_ref[...]
    n2 = n2_ref[...]
    pos_ref[:, 0] = jnp.sum(jnp.abs(n1 * r - n2), axis=1)
    nh_ref[:, 0] = jnp.sum(jnp.abs(hn_ref[...] * r - n2), axis=1)
    nt_ref[:, 0] = jnp.sum(jnp.abs(n1 * r - tn_ref[...]), axis=1)


def _tc_scores(table, idx, n1, n2, hn, tn):
    idx2d = idx.reshape(E, 1)
    edge_spec = pl.BlockSpec((B, D), lambda i: (i, 0))
    score_spec = pl.BlockSpec((B, 1), lambda i: (i, 0))
    pos, nh, nt = pl.pallas_call(
        _tc_score_kernel,
        grid=(E // B,),
        in_specs=[
            pl.BlockSpec((B, 1), lambda i: (i, 0)),
            pl.BlockSpec((R, D), lambda i: (0, 0)),
            edge_spec, edge_spec, edge_spec, edge_spec,
        ],
        out_specs=[score_spec, score_spec, score_spec],
        out_shape=[
            jax.ShapeDtypeStruct((E, 1), jnp.float32),
            jax.ShapeDtypeStruct((E, 1), jnp.float32),
            jax.ShapeDtypeStruct((E, 1), jnp.float32),
        ],
    )(idx2d, table, n1, n2, hn, tn)
    return pos.reshape(E), nh.reshape(E), nt.reshape(E)


def kernel(update_rel_embed, edge_type, node1_encoder_result,
           node2_encoder_result, head_neg_encoder_result,
           tail_neg_encoder_result):
    idx = edge_type.astype(jnp.int32)
    r = _sc_gather(update_rel_embed, idx)
    pos, nh, nt = _tc_scores(update_rel_embed, idx, node1_encoder_result,
                             node2_encoder_result, head_neg_encoder_result,
                             tail_neg_encoder_result)
    return (pos, nh, nt, r)
